# baseline (speedup 1.0000x reference)
import jax
import jax.numpy as jnp
from jax import lax
from jax._src.pallas import mpmd
from jax.experimental import pallas as pl
from jax.experimental.pallas import tpu as pltpu
from jax.experimental.pallas import tpu_sc as plsc

BLOCK = 8192
EMBED = 1024

_info = plsc.get_sparse_core_info()
_NC, _NS = _info.num_cores, _info.num_subcores
_NW = _NC * _NS

_SCS_ROWS = 1280
_TEC_ROWS = BLOCK - _NC * _SCS_ROWS
_ROWS_PER_W = _TEC_ROWS // _NW
_CHUNK = 16
_NSTEPS = _ROWS_PER_W // _CHUNK
_NBUF = 6
_AHEAD = 4

_SCS_CHUNK = 64
_SCS_STEPS = _SCS_ROWS // _SCS_CHUNK


def _tec_fn(pe_hbm, out_hbm, spmem):
    del spmem

    def body(*rest):
        bufs = rest[:_NBUF]
        sem_in, sem_out = rest[_NBUF], rest[_NBUF + 1]
        wid = lax.axis_index("s") * _NC + lax.axis_index("c")
        base = wid * _ROWS_PER_W

        def start_in(i):
            return pltpu.async_copy(
                pe_hbm.at[pl.ds(base + i * _CHUNK, _CHUNK)],
                bufs[i % _NBUF], sem_in)

        def start_out(i):
            return pltpu.async_copy(
                bufs[i % _NBUF],
                out_hbm.at[pl.ds(base + i * _CHUNK, _CHUNK)], sem_out)

        copies_in = [None] * _NSTEPS
        copies_out = [None] * _NSTEPS
        out_waited = [False] * _NSTEPS
        for i in range(_AHEAD):
            copies_in[i] = start_in(i)
        for i in range(_NSTEPS):
            copies_in[i].wait()
            copies_out[i] = start_out(i)
            j = i + _AHEAD
            if j < _NSTEPS:
                prev = j - _NBUF
                if prev >= 0:
                    copies_out[prev].wait()
                    out_waited[prev] = True
                copies_in[j] = start_in(j)
        for i in range(_NSTEPS):
            if not out_waited[i]:
                copies_out[i].wait()

    pl.run_scoped(
        body,
        *([pltpu.VMEM((_CHUNK, EMBED), jnp.float32)] * _NBUF),
        pltpu.SemaphoreType.DMA,
        pltpu.SemaphoreType.DMA,
    )


def _scs_fn(pe_hbm, out_hbm, spmem):
    def body(sem_in, sem_out):
        c = lax.axis_index("c")
        base = _TEC_ROWS + c * _SCS_ROWS

        def start_in(i):
            return pltpu.async_copy(
                pe_hbm.at[pl.ds(base + i * _SCS_CHUNK, _SCS_CHUNK)],
                spmem.at[i % 4], sem_in)

        def start_out(i):
            return pltpu.async_copy(
                spmem.at[i % 4],
                out_hbm.at[pl.ds(base + i * _SCS_CHUNK, _SCS_CHUNK)], sem_out)

        copies_in = [None] * _SCS_STEPS
        copies_out = [None] * _SCS_STEPS
        out_waited = [False] * _SCS_STEPS
        copies_in[0] = start_in(0)
        copies_in[1] = start_in(1)
        for i in range(_SCS_STEPS):
            copies_in[i].wait()
            copies_out[i] = start_out(i)
            j = i + 2
            if j < _SCS_STEPS:
                prev = j - 4
                if prev >= 0:
                    copies_out[prev].wait()
                    out_waited[prev] = True
                copies_in[j] = start_in(j)
        for i in range(_SCS_STEPS):
            if not out_waited[i]:
                copies_out[i].wait()

    pl.run_scoped(body, pltpu.SemaphoreType.DMA, pltpu.SemaphoreType.DMA)


def _sc_copy(pe):
    vec_mesh = plsc.VectorSubcoreMesh(core_axis_name="c", subcore_axis_name="s")
    scs_mesh = plsc.ScalarSubcoreMesh(axis_name="c")
    return mpmd.mpmd_map(
        [(scs_mesh, _scs_fn), (vec_mesh, _tec_fn)],
        out_types=[jax.ShapeDtypeStruct((BLOCK, EMBED), jnp.float32)],
        scratch_types=[
            pltpu.VMEM_SHARED((4, _SCS_CHUNK, EMBED), jnp.float32),
        ],
    )(pe)[0]


def kernel(x, pe):
    return _sc_copy(pe)


# speedup vs baseline: 1.0254x; 1.0254x over previous
import jax
import jax.numpy as jnp
from jax import lax
from jax._src.pallas import mpmd
from jax.experimental import pallas as pl
from jax.experimental.pallas import tpu as pltpu
from jax.experimental.pallas import tpu_sc as plsc

BLOCK = 8192
EMBED = 1024

_info = plsc.get_sparse_core_info()
_NC, _NS = _info.num_cores, _info.num_subcores
_NW = _NC * _NS

_SCS_ROWS = 1024
_TEC_ROWS = BLOCK - _NC * _SCS_ROWS
_ROWS_PER_W = _TEC_ROWS // _NW
_CHUNK = 16
_NSTEPS = _ROWS_PER_W // _CHUNK
_NBUF = 6
_AHEAD = 4

_SCS_CHUNK = 64
_SCS_STEPS = _SCS_ROWS // _SCS_CHUNK


def _tec_fn(pe_hbm, out_hbm, spmem):
    del spmem

    def body(*rest):
        bufs = rest[:_NBUF]
        sem_in, sem_out = rest[_NBUF], rest[_NBUF + 1]
        wid = lax.axis_index("s") * _NC + lax.axis_index("c")
        base = wid * _ROWS_PER_W

        def start_in(i):
            return pltpu.async_copy(
                pe_hbm.at[pl.ds(base + i * _CHUNK, _CHUNK)],
                bufs[i % _NBUF], sem_in)

        def start_out(i):
            return pltpu.async_copy(
                bufs[i % _NBUF],
                out_hbm.at[pl.ds(base + i * _CHUNK, _CHUNK)], sem_out)

        copies_in = [None] * _NSTEPS
        copies_out = [None] * _NSTEPS
        out_waited = [False] * _NSTEPS
        for i in range(_AHEAD):
            copies_in[i] = start_in(i)
        for i in range(_NSTEPS):
            copies_in[i].wait()
            copies_out[i] = start_out(i)
            j = i + _AHEAD
            if j < _NSTEPS:
                prev = j - _NBUF
                if prev >= 0:
                    copies_out[prev].wait()
                    out_waited[prev] = True
                copies_in[j] = start_in(j)
        for i in range(_NSTEPS):
            if not out_waited[i]:
                copies_out[i].wait()

    pl.run_scoped(
        body,
        *([pltpu.VMEM((_CHUNK, EMBED), jnp.float32)] * _NBUF),
        pltpu.SemaphoreType.DMA,
        pltpu.SemaphoreType.DMA,
    )


def _scs_fn(pe_hbm, out_hbm, spmem):
    def body(sem_in, sem_out):
        c = lax.axis_index("c")
        base = _TEC_ROWS + c * _SCS_ROWS

        def start_in(i):
            return pltpu.async_copy(
                pe_hbm.at[pl.ds(base + i * _SCS_CHUNK, _SCS_CHUNK)],
                spmem.at[i % 4], sem_in)

        def start_out(i):
            return pltpu.async_copy(
                spmem.at[i % 4],
                out_hbm.at[pl.ds(base + i * _SCS_CHUNK, _SCS_CHUNK)], sem_out)

        copies_in = [None] * _SCS_STEPS
        copies_out = [None] * _SCS_STEPS
        out_waited = [False] * _SCS_STEPS
        copies_in[0] = start_in(0)
        copies_in[1] = start_in(1)
        for i in range(_SCS_STEPS):
            copies_in[i].wait()
            copies_out[i] = start_out(i)
            j = i + 2
            if j < _SCS_STEPS:
                prev = j - 4
                if prev >= 0:
                    copies_out[prev].wait()
                    out_waited[prev] = True
                copies_in[j] = start_in(j)
        for i in range(_SCS_STEPS):
            if not out_waited[i]:
                copies_out[i].wait()

    pl.run_scoped(body, pltpu.SemaphoreType.DMA, pltpu.SemaphoreType.DMA)


def _sc_copy(pe):
    vec_mesh = plsc.VectorSubcoreMesh(core_axis_name="c", subcore_axis_name="s")
    scs_mesh = plsc.ScalarSubcoreMesh(axis_name="c")
    return mpmd.mpmd_map(
        [(scs_mesh, _scs_fn), (vec_mesh, _tec_fn)],
        out_types=[jax.ShapeDtypeStruct((BLOCK, EMBED), jnp.float32)],
        scratch_types=[
            pltpu.VMEM_SHARED((4, _SCS_CHUNK, EMBED), jnp.float32),
        ],
    )(pe)[0]


def kernel(x, pe):
    return _sc_copy(pe)
